# baseline (speedup 1.0000x reference)
import functools

import jax
import jax.numpy as jnp
from jax import lax
from jax.experimental import pallas as pl
from jax.experimental.pallas import tpu as pltpu
from jax.experimental.pallas import tpu_sc as plsc

_B, _TM1, _V = 8, 2048, 4096
_NTOK = _B * _TM1
_TB = 1024
_H = _TB // 2
_NB = _NTOK // _TB
_RPB = _TM1 // _TB

_NW = 32
_TPW = _NTOK // _NW
_L = 16
_ROWW = 16
_NCHUNK = _TPW // 128
_NVEC = _TPW // _L


def _half_lse_sum(x):
    m = jnp.max(x, axis=-1, keepdims=True)
    s = jnp.sum(jnp.exp(x - m), axis=-1, keepdims=True)
    return jnp.sum(jnp.log(s) + m)


def _lse_kernel(length_ref, xa_ref, xb_ref, out_ref, acc_ref):
    i = pl.program_id(0)

    @pl.when(i == 0)
    def _():
        acc_ref[0] = 0.0
        acc_ref[1] = 0.0

    lse = _half_lse_sum(xa_ref[0]) + _half_lse_sum(xb_ref[0])
    w = jnp.where(length_ref[i // _RPB] > 0, 1.0, 0.0)
    acc_ref[0] += w * lse
    acc_ref[1] += w * _TB

    @pl.when(i == _NB - 1)
    def _():
        out_ref[0, 0] = acc_ref[0]
        out_ref[1, 0] = acc_ref[1]


def _tc_lse_sums(x, length):
    grid_spec = pltpu.PrefetchScalarGridSpec(
        num_scalar_prefetch=1,
        grid=(_NB,),
        in_specs=[
            pl.BlockSpec((1, _H, _V), lambda i, *_: (i, 0, 0)),
            pl.BlockSpec((1, _H, _V), lambda i, *_: (i, 1, 0)),
        ],
        out_specs=pl.BlockSpec((2, 1), lambda i, *_: (0, 0),
                               memory_space=pltpu.SMEM),
        scratch_shapes=[pltpu.SMEM((2,), jnp.float32)],
    )
    return pl.pallas_call(
        _lse_kernel,
        grid_spec=grid_spec,
        out_shape=jax.ShapeDtypeStruct((2, 1), jnp.float32),
    )(length, x, x)


def _sc_gather_kernel(x_hbm, tgt_hbm, out_hbm,
                      tgt_v, fidx_v, vals_v, acc_v, sem):
    wid = lax.axis_index("s") * 2 + lax.axis_index("c")
    base = wid * _TPW
    pltpu.sync_copy(tgt_hbm.at[pl.ds(base, _TPW)], tgt_v)

    def _mk_idx(c, carry):
        j16 = base + c * _L + lax.iota(jnp.int32, _L)
        t16 = tgt_v[pl.ds(c * _L, _L)]
        fidx_v[c // 8, pl.ds((c % 8) * _L, _L)] = j16 * _V + t16
        return carry

    lax.fori_loop(0, _NVEC, _mk_idx, 0)

    copies = []
    for c in range(_NCHUNK):
        copies.append(pltpu.make_async_copy(
            x_hbm.at[fidx_v.at[c]],
            vals_v.at[pl.ds(c * 128, 128)],
            sem))
    for cp in copies:
        cp.start()
    for cp in copies:
        cp.wait()

    def _pick(c, acc):
        return acc + vals_v[pl.ds(c * _L, _L)]

    acc = lax.fori_loop(0, _NVEC, _pick,
                        jnp.zeros((_L,), jnp.float32))
    acc_v[...] = acc
    pltpu.sync_copy(acc_v, out_hbm.at[wid])


def _sc_picked_sums(x2d, tgt_flat):
    mesh = plsc.VectorSubcoreMesh(core_axis_name="c", subcore_axis_name="s")
    k = functools.partial(
        pl.kernel, mesh=mesh,
        out_type=jax.ShapeDtypeStruct((_NW, _L), jnp.float32),
        scratch_types=[
            pltpu.VMEM((_TPW,), jnp.int32),
            pltpu.VMEM((_NCHUNK, 128), jnp.int32),
            pltpu.VMEM((_TPW,), jnp.float32),
            pltpu.VMEM((_L,), jnp.float32),
            pltpu.SemaphoreType.DMA,
        ],
    )(_sc_gather_kernel)
    return k(x2d, tgt_flat)


def _combine_kernel(lse_ref, picked_ref, length_ref, out_ref):
    p = picked_ref[...]
    total_picked = 0.0
    for b in range(_B):
        w = jnp.where(length_ref[b] > 0, 1.0, 0.0)
        total_picked += w * jnp.sum(p[4 * b:4 * b + 4, :])
    total = lse_ref[0, 0] - total_picked
    count = jnp.maximum(lse_ref[1, 0], 1.0)
    out_ref[0, 0] = total / count


def kernel(input, target, length):
    x = input.reshape(_NB, _TB, _V)
    x1d = input.reshape(_NTOK * _V)
    tgt_flat = target[:, 1:].reshape(_NTOK)

    lse_sums = _tc_lse_sums(x, length)
    picked = _sc_picked_sums(x1d, tgt_flat)

    out = pl.pallas_call(
        _combine_kernel,
        in_specs=[
            pl.BlockSpec(memory_space=pltpu.SMEM),
            pl.BlockSpec(),
            pl.BlockSpec(memory_space=pltpu.SMEM),
        ],
        out_specs=pl.BlockSpec(memory_space=pltpu.SMEM),
        out_shape=jax.ShapeDtypeStruct((1, 1), jnp.float32),
    )(lse_sums, picked, length)
    return out[0, 0]


# speedup vs baseline: 1.1117x; 1.1117x over previous
import jax
import jax.numpy as jnp
from jax.experimental import pallas as pl
from jax.experimental.pallas import tpu as pltpu

_B, _TM1, _V = 8, 2048, 4096
_TB = 1024
_H = _TB // 2
_NB = (_B * _TM1) // _TB
_RPB = _TM1 // _TB
_RT = 32
_NCH = _V // 128


def _half_nll_sum(x_ref, t_ref):
    lane = jax.lax.broadcasted_iota(jnp.int32, (_RT, 128), 1)

    def tile(rt, acc):
        r0 = rt * _RT
        tgt = t_ref[0, pl.ds(r0, _RT), :]
        m = jnp.full((_RT, 128), -jnp.inf, jnp.float32)
        p = jnp.zeros((_RT, 128), jnp.float32)
        for c in range(_NCH):
            xc = x_ref[0, pl.ds(r0, _RT), c * 128:(c + 1) * 128]
            m = jnp.maximum(m, xc)
            p = p + jnp.where(lane + c * 128 == tgt, xc, 0.0)
        mrow = jnp.max(m, axis=1, keepdims=True)
        picked = jnp.sum(p, axis=1, keepdims=True)
        s = jnp.zeros((_RT, 1), jnp.float32)
        for c in range(_NCH):
            xc = x_ref[0, pl.ds(r0, _RT), c * 128:(c + 1) * 128]
            s = s + jnp.sum(jnp.exp(xc - mrow), axis=1, keepdims=True)
        return acc + jnp.sum(jnp.log(s) + mrow - picked)

    return jax.lax.fori_loop(0, _H // _RT, tile, jnp.float32(0.0))


def _nll_kernel(length_ref, xa_ref, xb_ref, ta_ref, tb_ref, out_ref, acc_ref):
    i = pl.program_id(0)

    @pl.when(i == 0)
    def _():
        acc_ref[0] = 0.0
        acc_ref[1] = 0.0

    nll = _half_nll_sum(xa_ref, ta_ref) + _half_nll_sum(xb_ref, tb_ref)
    w = jnp.where(length_ref[i // _RPB] > 0, 1.0, 0.0)
    acc_ref[0] += w * nll
    acc_ref[1] += w * _TB

    @pl.when(i == _NB - 1)
    def _():
        out_ref[0, 0] = acc_ref[0] / jnp.maximum(acc_ref[1], 1.0)


def kernel(input, target, length):
    x = input.reshape(_NB, _TB, _V)
    tgt = target[:, 1:].reshape(_NB, _TB, 1)
    grid_spec = pltpu.PrefetchScalarGridSpec(
        num_scalar_prefetch=1,
        grid=(_NB,),
        in_specs=[
            pl.BlockSpec((1, _H, _V), lambda i, *_: (i, 0, 0)),
            pl.BlockSpec((1, _H, _V), lambda i, *_: (i, 1, 0)),
            pl.BlockSpec((1, _H, 1), lambda i, *_: (i, 0, 0)),
            pl.BlockSpec((1, _H, 1), lambda i, *_: (i, 1, 0)),
        ],
        out_specs=pl.BlockSpec((1, 1), lambda i, *_: (0, 0),
                               memory_space=pltpu.SMEM),
        scratch_shapes=[pltpu.SMEM((2,), jnp.float32)],
    )
    out = pl.pallas_call(
        _nll_kernel,
        grid_spec=grid_spec,
        out_shape=jax.ShapeDtypeStruct((1, 1), jnp.float32),
    )(length, x, x, tgt, tgt)
    return out[0, 0]


# speedup vs baseline: 2.6740x; 2.4052x over previous
import jax
import jax.numpy as jnp
from jax.experimental import pallas as pl
from jax.experimental.pallas import tpu as pltpu

_B, _TM1, _V = 8, 2048, 4096
_TB = 1024
_NQ = 4
_H = _TB // _NQ
_NB = (_B * _TM1) // _TB
_RPB = _TM1 // _TB


def _part_nll_sum(x, tgt):
    m = jnp.max(x, axis=-1, keepdims=True)
    s = jnp.sum(jnp.exp(x - m), axis=-1, keepdims=True)
    iota = jax.lax.broadcasted_iota(jnp.int32, (_H, _V), 1)
    picked = jnp.sum(jnp.where(iota == tgt, x, 0.0), axis=-1, keepdims=True)
    return jnp.sum(jnp.log(s) + m - picked)


def _nll_kernel(length_ref, x0, x1, x2, x3, t0, t1, t2, t3, out_ref, acc_ref):
    i = pl.program_id(0)

    @pl.when(i == 0)
    def _():
        acc_ref[0] = 0.0
        acc_ref[1] = 0.0

    nll = (_part_nll_sum(x0[0], t0[0]) + _part_nll_sum(x1[0], t1[0])
           + _part_nll_sum(x2[0], t2[0]) + _part_nll_sum(x3[0], t3[0]))
    w = jnp.where(length_ref[i // _RPB] > 0, 1.0, 0.0)
    acc_ref[0] += w * nll
    acc_ref[1] += w * _TB

    @pl.when(i == _NB - 1)
    def _():
        out_ref[0, 0] = acc_ref[0] / jnp.maximum(acc_ref[1], 1.0)


def kernel(input, target, length):
    x = input.reshape(_NB, _TB, _V)
    tgt = target[:, 1:].reshape(_NB, _TB, 1)

    def xspec(q):
        return pl.BlockSpec((1, _H, _V), lambda i, *_: (i, q, 0))

    def tspec(q):
        return pl.BlockSpec((1, _H, 1), lambda i, *_: (i, q, 0))

    grid_spec = pltpu.PrefetchScalarGridSpec(
        num_scalar_prefetch=1,
        grid=(_NB,),
        in_specs=[xspec(0), xspec(1), xspec(2), xspec(3),
                  tspec(0), tspec(1), tspec(2), tspec(3)],
        out_specs=pl.BlockSpec((1, 1), lambda i, *_: (0, 0),
                               memory_space=pltpu.SMEM),
        scratch_shapes=[pltpu.SMEM((2,), jnp.float32)],
    )
    out = pl.pallas_call(
        _nll_kernel,
        grid_spec=grid_spec,
        out_shape=jax.ShapeDtypeStruct((1, 1), jnp.float32),
    )(length, x, x, x, x, tgt, tgt, tgt, tgt)
    return out[0, 0]


# speedup vs baseline: 2.7077x; 1.0126x over previous
import jax
import jax.numpy as jnp
from jax.experimental import pallas as pl
from jax.experimental.pallas import tpu as pltpu

_B, _TM1, _V = 8, 2048, 4096
_TB = 1024
_H = _TB // 2
_NB = (_B * _TM1) // _TB
_RPB = _TM1 // _TB


def _half_nll_sum(x, tgt):
    m = jnp.max(x, axis=-1, keepdims=True)
    e = jnp.exp(x - m)
    s = jnp.sum(e, axis=-1, keepdims=True)
    iota = jax.lax.broadcasted_iota(jnp.int32, (_H, _V), 1)
    pe = jnp.sum(jnp.where(iota == tgt, e, 0.0), axis=-1, keepdims=True)
    return jnp.sum(jnp.log(s) - jnp.log(pe))


def _nll_kernel(length_ref, xa_ref, xb_ref, ta_ref, tb_ref, out_ref, acc_ref):
    i = pl.program_id(0)

    @pl.when(i == 0)
    def _():
        acc_ref[0] = 0.0
        acc_ref[1] = 0.0

    nll = _half_nll_sum(xa_ref[0], ta_ref[0]) + _half_nll_sum(xb_ref[0], tb_ref[0])
    w = jnp.where(length_ref[i // _RPB] > 0, 1.0, 0.0)
    acc_ref[0] += w * nll
    acc_ref[1] += w * _TB

    @pl.when(i == _NB - 1)
    def _():
        out_ref[0, 0] = acc_ref[0] / jnp.maximum(acc_ref[1], 1.0)


def kernel(input, target, length):
    x = input.reshape(_NB, _TB, _V)
    tgt = target[:, 1:].reshape(_NB, _TB, 1)
    grid_spec = pltpu.PrefetchScalarGridSpec(
        num_scalar_prefetch=1,
        grid=(_NB,),
        in_specs=[
            pl.BlockSpec((1, _H, _V), lambda i, *_: (i, 0, 0)),
            pl.BlockSpec((1, _H, _V), lambda i, *_: (i, 1, 0)),
            pl.BlockSpec((1, _H, 1), lambda i, *_: (i, 0, 0)),
            pl.BlockSpec((1, _H, 1), lambda i, *_: (i, 1, 0)),
        ],
        out_specs=pl.BlockSpec((1, 1), lambda i, *_: (0, 0),
                               memory_space=pltpu.SMEM),
        scratch_shapes=[pltpu.SMEM((2,), jnp.float32)],
    )
    out = pl.pallas_call(
        _nll_kernel,
        grid_spec=grid_spec,
        out_shape=jax.ShapeDtypeStruct((1, 1), jnp.float32),
    )(length, x, x, tgt, tgt)
    return out[0, 0]


# speedup vs baseline: 2.7166x; 1.0033x over previous
import jax
import jax.numpy as jnp
from jax.experimental import pallas as pl
from jax.experimental.pallas import tpu as pltpu

_B, _TM1, _V = 8, 2048, 4096
_TB = 1024
_H = _TB // 2
_NB = (_B * _TM1) // _TB
_RPB = _TM1 // _TB


def _half_nll_sum(x, tgt):
    m = jnp.max(x, axis=-1, keepdims=True)
    s = jnp.sum(jnp.exp(x - m), axis=-1, keepdims=True)
    iota = jax.lax.broadcasted_iota(jnp.int32, (_H, _V), 1)
    picked = jnp.sum(jnp.where(iota == tgt, x, 0.0), axis=-1, keepdims=True)
    return jnp.sum(jnp.log(s) + m - picked)


def _nll_kernel(length_ref, xa_ref, xb_ref, ta_ref, tb_ref, out_ref, acc_ref):
    i = pl.program_id(0)

    @pl.when(i == 0)
    def _():
        acc_ref[0] = 0.0
        acc_ref[1] = 0.0

    nll = _half_nll_sum(xa_ref[0], ta_ref[0]) + _half_nll_sum(xb_ref[0], tb_ref[0])
    w = jnp.where(length_ref[i // _RPB] > 0, 1.0, 0.0)
    acc_ref[0] += w * nll
    acc_ref[1] += w * _TB

    @pl.when(i == _NB - 1)
    def _():
        out_ref[0, 0] = acc_ref[0] / jnp.maximum(acc_ref[1], 1.0)


def kernel(input, target, length):
    x = input.reshape(_NB, _TB, _V)
    tgt = target[:, 1:].reshape(_NB, _TB, 1)
    grid_spec = pltpu.PrefetchScalarGridSpec(
        num_scalar_prefetch=1,
        grid=(_NB,),
        in_specs=[
            pl.BlockSpec((1, _H, _V), lambda i, *_: (i, 0, 0)),
            pl.BlockSpec((1, _H, _V), lambda i, *_: (i, 1, 0)),
            pl.BlockSpec((1, _H, 1), lambda i, *_: (i, 0, 0)),
            pl.BlockSpec((1, _H, 1), lambda i, *_: (i, 1, 0)),
        ],
        out_specs=pl.BlockSpec((1, 1), lambda i, *_: (0, 0),
                               memory_space=pltpu.SMEM),
        scratch_shapes=[pltpu.SMEM((2,), jnp.float32)],
    )
    out = pl.pallas_call(
        _nll_kernel,
        grid_spec=grid_spec,
        out_shape=jax.ShapeDtypeStruct((1, 1), jnp.float32),
    )(length, x, x, tgt, tgt)
    return out[0, 0]
